# baseline (speedup 1.0000x reference)
import functools

import jax
import jax.numpy as jnp
from jax import lax
from jax.experimental import pallas as pl
from jax.experimental.pallas import tpu as pltpu, tpu_sc as plsc

B = 8
C = 384
CR = 24
K = 192
H = 224
W = 224
HW = H * W
HSC = 56
MHCH = 56
NMH = (H - HSC) // MHCH
GHCH = 16
NGH = H // GHCH

NW = 32
SLABS_PER_W = B * HSC // NW
WCH = 56
NCH = W // WCH
NACC = C // 16

_sc_mesh = plsc.VectorSubcoreMesh(core_axis_name="c", subcore_axis_name="s")


@functools.partial(
    pl.kernel,
    mesh=_sc_mesh,
    out_type=jax.ShapeDtypeStruct((NW * C,), jnp.float32),
    scratch_types=[
        pltpu.VMEM((WCH * C,), jnp.float32),
        pltpu.VMEM((C,), jnp.float32),
        pltpu.SemaphoreType.DMA,
    ],
)
def _sc_mean(x_hbm, o_hbm, buf, accm, sem):
    cc = lax.axis_index("c")
    ss = lax.axis_index("s")
    wid = ss * 2 + cc
    b = wid // 4
    hbase = (wid % 4) * SLABS_PER_W

    zero = jnp.zeros((16,), jnp.float32)
    for v in range(NACC):
        accm[pl.ds(v * 16, 16)] = zero

    def chunk_body(i, carry):
        line = i // NCH
        wq = i % NCH
        h = hbase + line
        off = (((b * H + h) * W) + wq * WCH) * C
        pltpu.async_copy(x_hbm.at[pl.ds(off, WCH * C)], buf, sem).wait()
        acc = [zero] * NACC
        for w in range(WCH):
            base = w * C
            for v in range(NACC):
                acc[v] = acc[v] + buf[pl.ds(base + v * 16, 16)]
        for v in range(NACC):
            accm[pl.ds(v * 16, 16)] += acc[v]
        return carry

    lax.fori_loop(0, SLABS_PER_W * NCH, chunk_body, 0, unroll=False)
    orow = (wid % 4) * B + b
    pltpu.sync_copy(accm, o_hbm.at[pl.ds(orow * C, C)])


def _mean_body(x_ref, o_ref):
    h = pl.program_id(1)
    s = jnp.sum(x_ref[...], axis=(1, 2)).reshape(1, 1, C)

    @pl.when(h == 0)
    def _():
        o_ref[...] = s

    @pl.when(h > 0)
    def _():
        o_ref[...] += s


def _logits_body(ys_ref, sc_ref, w1t_ref, b1_ref, w2t_ref, b2_ref, z_ref):
    scp = (sc_ref[0:B, :] + sc_ref[B:2 * B, :]
           + sc_ref[2 * B:3 * B, :] + sc_ref[3 * B:4 * B, :])
    y = (ys_ref[...].reshape(B, C) + scp) / float(HW)
    h = jnp.dot(y, w1t_ref[...], preferred_element_type=jnp.float32)
    h = jnp.maximum(h + b1_ref[...], 0.0)
    z = jnp.dot(h, w2t_ref[...], preferred_element_type=jnp.float32)
    z_ref[...] = z + b2_ref[...]


def _select_body(a_ref, at_ref, g_ref):
    row_i = lax.broadcasted_iota(jnp.int32, (C, C), 0)
    col_i = lax.broadcasted_iota(jnp.int32, (C, C), 1)
    lte = (row_i <= col_i).astype(jnp.float32)
    jKC = lax.broadcasted_iota(jnp.int32, (K, C), 0)
    for b in range(B):
        vrow = a_ref[pl.ds(b, 1), :]
        vcol = at_ref[:, pl.ds(b, 1)]
        vr = jnp.broadcast_to(vrow, (C, C))
        vc = jnp.broadcast_to(vcol, (C, C))
        gt = (vc > vr) | ((vc == vr) & (row_i < col_i))
        rank = jnp.sum(gt.astype(jnp.float32), axis=0, keepdims=True)
        maskb = rank < float(K)
        pos = jnp.dot(maskb.astype(jnp.float32), lte,
                      preferred_element_type=jnp.float32)
        posi = pos.astype(jnp.int32) - 1
        oh = (jnp.broadcast_to(posi, (K, C)) == jKC) & jnp.broadcast_to(
            maskb, (K, C))
        gb = jnp.where(oh, jnp.broadcast_to(vrow, (K, C)), 0.0)
        g_ref[pl.ds(b, 1), :, :] = gb.reshape(1, K, C)


def _gather_body(g_ref, x_ref, o_ref):
    g = g_ref[0]
    for hh in range(GHCH):
        xrow = x_ref[0, hh]
        o = lax.dot_general(g, xrow, (((1,), (1,)), ((), ())),
                            preferred_element_type=jnp.float32)
        o_ref[0, :, hh, :] = o


def kernel(x, W1, b1, W2, b2):
    xt = jnp.transpose(x, (0, 2, 3, 1))

    sc_part = _sc_mean(xt.reshape(-1)).reshape(NW, C)

    ysum = pl.pallas_call(
        _mean_body,
        grid=(B, NMH),
        in_specs=[pl.BlockSpec((1, MHCH, W, C), lambda b, h: (b, h + 1, 0, 0))],
        out_specs=pl.BlockSpec((1, 1, C), lambda b, h: (b, 0, 0)),
        out_shape=jax.ShapeDtypeStruct((B, 1, C), jnp.float32),
    )(xt)

    z = pl.pallas_call(
        _logits_body,
        out_shape=jax.ShapeDtypeStruct((B, C), jnp.float32),
    )(ysum, sc_part, W1.T, b1.reshape(1, CR), W2.T, b2.reshape(1, C))

    a = jax.nn.sigmoid(z)

    G = pl.pallas_call(
        _select_body,
        out_shape=jax.ShapeDtypeStruct((B, K, C), jnp.float32),
    )(a, a.T)

    out = pl.pallas_call(
        _gather_body,
        grid=(B, NGH),
        in_specs=[pl.BlockSpec((1, K, C), lambda b, h: (b, 0, 0)),
                  pl.BlockSpec((1, GHCH, W, C), lambda b, h: (b, h, 0, 0))],
        out_specs=pl.BlockSpec((1, K, GHCH, W), lambda b, h: (b, 0, h, 0)),
        out_shape=jax.ShapeDtypeStruct((B, K, H, W), jnp.float32),
    )(G, xt)

    return out
